# Initial kernel scaffold; baseline (speedup 1.0000x reference)
import jax
import jax.numpy as jnp
from jax.experimental import pallas as pl


def kernel(x, edge_index, batch, num_graphs, W_a1, b_a1, W_a2, b_a2, W_x1, b_x1, W_x2, b_x2, W_lin, b_lin):
    raise NotImplementedError("write your pallas kernel here")


# speedup vs baseline: 9.2691x; 9.2691x over previous
import functools

import jax
import jax.numpy as jnp
from jax import lax
from jax.experimental import pallas as pl
from jax.experimental.pallas import tpu as pltpu
from jax.experimental.pallas import tpu_sc as plsc

N = 10000
NPAD = 10240
ROWS = NPAD // 16
E = 320000
EPAD = 323584
K = 128
G = 16
D0, D1, D2, DOUT = 128, 256, 128, 10
F32 = jnp.float32

def _mesh():
    return plsc.VectorSubcoreMesh(
        core_axis_name="c", subcore_axis_name="s", num_cores=2,
        num_subcores=16)


def _sc_deg(dst_p, z128, ones128):
    @functools.partial(
        pl.kernel,
        out_type=jax.ShapeDtypeStruct((2 * NPAD, 128), F32),
        mesh=_mesh(),
        scratch_types=[
            pltpu.VMEM((K,), jnp.int32),
            pltpu.VMEM((K, 128), F32),
            pltpu.VMEM_SHARED((NPAD, 128), F32),
        ],
    )
    def k(dst_hbm, z_hbm, ones_hbm, out_hbm, dstbuf, onesbuf, acc):
        cid = lax.axis_index("c")
        sid = lax.axis_index("s")
        row_lo = sid * ROWS
        pltpu.sync_copy(z_hbm, acc.at[pl.ds(row_lo, ROWS)])
        pltpu.sync_copy(ones_hbm, onesbuf)
        plsc.subcore_barrier()
        ebase = (cid * 16 + sid) * (EPAD // 32)

        def body(j, carry):
            pltpu.sync_copy(dst_hbm.at[pl.ds(ebase + j * K, K)], dstbuf)
            pltpu.sync_copy(onesbuf, acc.at[dstbuf], add=True)
            return carry

        lax.fori_loop(0, EPAD // 32 // K, body, 0)
        plsc.subcore_barrier()
        pltpu.sync_copy(acc.at[pl.ds(row_lo, ROWS)],
                        out_hbm.at[pl.ds(cid * NPAD + row_lo, ROWS)])

    return k(dst_p, z128, ones128)


def _sc_agg(t0, t1, src_p, dst_p, z128, split_edges):
    nb = (EPAD // 32 if split_edges else EPAD // 16) // K

    @functools.partial(
        pl.kernel,
        out_type=jax.ShapeDtypeStruct((2 * NPAD, 128), F32),
        mesh=_mesh(),
        scratch_types=[
            pltpu.VMEM((K,), jnp.int32),
            pltpu.VMEM((K,), jnp.int32),
            pltpu.VMEM((K, 128), F32),
            pltpu.VMEM_SHARED((NPAD, 128), F32),
            pltpu.SemaphoreType.DMA,
        ],
    )
    def k(t0_hbm, t1_hbm, src_hbm, dst_hbm, z_hbm, out_hbm,
          srcbuf, dstbuf, rows, acc, sem):
        cid = lax.axis_index("c")
        sid = lax.axis_index("s")
        row_lo = sid * ROWS
        pltpu.sync_copy(z_hbm, acc.at[pl.ds(row_lo, ROWS)])
        plsc.subcore_barrier()
        if split_edges:
            ebase = (cid * 16 + sid) * (EPAD // 32)
        else:
            ebase = sid * (EPAD // 16)

        def body(j, carry):
            eoff = ebase + j * K
            pltpu.sync_copy(src_hbm.at[pl.ds(eoff, K)], srcbuf)
            pltpu.sync_copy(dst_hbm.at[pl.ds(eoff, K)], dstbuf)

            @pl.when(cid == 0)
            def _():
                pltpu.async_copy(t0_hbm.at[srcbuf], rows, sem).wait()

            @pl.when(cid == 1)
            def _():
                pltpu.async_copy(t1_hbm.at[srcbuf], rows, sem).wait()

            pltpu.sync_copy(rows, acc.at[dstbuf], add=True)
            return carry

        lax.fori_loop(0, nb, body, 0)
        plsc.subcore_barrier()
        pltpu.sync_copy(acc.at[pl.ds(row_lo, ROWS)],
                        out_hbm.at[pl.ds(cid * NPAD + row_lo, ROWS)])

    return k(t0, t1, src_p, dst_p, z128)


_NBLK = NPAD // ROWS


def _tc_prep(degp, x_p):
    def body(d0, d1, x_ref, dinv_ref, u_ref):
        deg = d0[:, 0:1] + d1[:, 0:1] + 2.0
        dinv = lax.rsqrt(deg)
        dinv_ref[...] = dinv
        u_ref[...] = dinv * x_ref[...]

    return pl.pallas_call(
        body,
        grid=(_NBLK,),
        in_specs=[
            pl.BlockSpec((ROWS, 128), lambda i: (i, 0)),
            pl.BlockSpec((ROWS, 128), lambda i: (_NBLK + i, 0)),
            pl.BlockSpec((ROWS, 128), lambda i: (i, 0)),
        ],
        out_specs=[
            pl.BlockSpec((ROWS, 1), lambda i: (i, 0)),
            pl.BlockSpec((ROWS, 128), lambda i: (i, 0)),
        ],
        out_shape=[
            jax.ShapeDtypeStruct((NPAD, 1), F32),
            jax.ShapeDtypeStruct((NPAD, 128), F32),
        ],
    )(degp, degp, x_p)


def _tc_mid(agg1, u, dinv, W_a1, b_a1, W_x1, b_x1, W_a2, W_x2):
    def body(p0, p1, u_ref, dinv_ref, wa1, ba1, wx1, bx1, wa2, wx2,
             ua_ref, ux_ref):
        dv = dinv_ref[...]
        px = dv * (p0[...] + p1[...] + 2.0 * u_ref[...])
        a1 = jnp.maximum(
            jnp.dot(px, wa1[...], preferred_element_type=F32) + ba1[...], 0.0)
        x1 = jnp.maximum(
            jnp.dot(px, wx1[...], preferred_element_type=F32) + bx1[...], 0.0)
        ua_ref[...] = dv * jnp.dot(a1, wa2[...], preferred_element_type=F32)
        ux_ref[...] = dv * jnp.dot(x1, wx2[...], preferred_element_type=F32)

    full = lambda s: pl.BlockSpec(s, lambda i: tuple(0 for _ in s))
    return pl.pallas_call(
        body,
        grid=(_NBLK,),
        in_specs=[
            pl.BlockSpec((ROWS, 128), lambda i: (i, 0)),
            pl.BlockSpec((ROWS, 128), lambda i: (_NBLK + i, 0)),
            pl.BlockSpec((ROWS, 128), lambda i: (i, 0)),
            pl.BlockSpec((ROWS, 1), lambda i: (i, 0)),
            full((D0, D1)), full((1, D1)), full((D0, D1)), full((1, D1)),
            full((D1, D2)), full((D1, D2)),
        ],
        out_specs=[
            pl.BlockSpec((ROWS, 128), lambda i: (i, 0)),
            pl.BlockSpec((ROWS, 128), lambda i: (i, 0)),
        ],
        out_shape=[
            jax.ShapeDtypeStruct((NPAD, 128), F32),
            jax.ShapeDtypeStruct((NPAD, 128), F32),
        ],
    )(agg1, agg1, u, dinv, W_a1, b_a1.reshape(1, D1), W_x1,
      b_x1.reshape(1, D1), W_a2, W_x2)


def _tc_f1(agg2, ua, ux, dinv, b_a2, b_x2):
    def body(q0, q1, ua_ref, ux_ref, dinv_ref, ba2, bx2,
             a2_ref, x2_ref, cmax_ref):
        i = pl.program_id(0)
        dv = dinv_ref[...]
        a2 = dv * (q0[...] + 2.0 * ua_ref[...]) + ba2[...]
        x2 = jnp.maximum(dv * (q1[...] + 2.0 * ux_ref[...]) + bx2[...], 0.0)
        rid = i * ROWS + lax.broadcasted_iota(jnp.int32, (ROWS, 1), 0)
        a2 = jnp.where(rid < N, a2, -1e30)
        a2_ref[...] = a2
        x2_ref[...] = x2

        @pl.when(i == 0)
        def _():
            cmax_ref[...] = jnp.full((8, 128), -1e30, F32)

        bm = jnp.max(a2, axis=0, keepdims=True)
        cmax_ref[0:1, :] = jnp.maximum(cmax_ref[0:1, :], bm)

    full = lambda s: pl.BlockSpec(s, lambda i: tuple(0 for _ in s))
    return pl.pallas_call(
        body,
        grid=(_NBLK,),
        in_specs=[
            pl.BlockSpec((ROWS, 128), lambda i: (i, 0)),
            pl.BlockSpec((ROWS, 128), lambda i: (_NBLK + i, 0)),
            pl.BlockSpec((ROWS, 128), lambda i: (i, 0)),
            pl.BlockSpec((ROWS, 128), lambda i: (i, 0)),
            pl.BlockSpec((ROWS, 1), lambda i: (i, 0)),
            full((1, D2)), full((1, D2)),
        ],
        out_specs=[
            pl.BlockSpec((ROWS, 128), lambda i: (i, 0)),
            pl.BlockSpec((ROWS, 128), lambda i: (i, 0)),
            full((8, 128)),
        ],
        out_shape=[
            jax.ShapeDtypeStruct((NPAD, 128), F32),
            jax.ShapeDtypeStruct((NPAD, 128), F32),
            jax.ShapeDtypeStruct((8, 128), F32),
        ],
    )(agg2, agg2, ua, ux, dinv, b_a2.reshape(1, D2), b_x2.reshape(1, D2))


def _tc_f2(a2, x2, cmax, batch_p):
    def body(a2_ref, x2_ref, cmax_ref, bat_ref, ct_out, cg_out, rt_out,
             Ct, Cg, SgT):
        i = pl.program_id(0)

        @pl.when(i == 0)
        def _():
            Ct[...] = jnp.zeros((128, 128), F32)
            Cg[...] = jnp.zeros((G, 128, 128), F32)
            SgT[...] = jnp.zeros((128, G), F32)

        U = jnp.exp(a2_ref[...] - cmax_ref[0:1, :])
        x2b = x2_ref[...]
        Ct[...] = Ct[...] + lax.dot_general(
            U, x2b, (((0,), (0,)), ((), ())), preferred_element_type=F32)
        oh = (bat_ref[...] ==
              lax.broadcasted_iota(jnp.int32, (1, G), 1)).astype(F32)
        SgT[...] = SgT[...] + lax.dot_general(
            U, oh, (((0,), (0,)), ((), ())), preferred_element_type=F32)
        for g in range(G):
            Ug = U * oh[:, g:g + 1]
            Cg[g] = Cg[g] + lax.dot_general(
                Ug, x2b, (((0,), (0,)), ((), ())), preferred_element_type=F32)

        @pl.when(i == _NBLK - 1)
        def _():
            st = jnp.sum(SgT[...], axis=1, keepdims=True)
            rt_out[...] = 1.0 / (st - SgT[...])
            ct_out[...] = Ct[...]
            cg_out[...] = Cg[...]

    full = lambda s: pl.BlockSpec(s, lambda i: tuple(0 for _ in s))
    return pl.pallas_call(
        body,
        grid=(_NBLK,),
        in_specs=[
            pl.BlockSpec((ROWS, 128), lambda i: (i, 0)),
            pl.BlockSpec((ROWS, 128), lambda i: (i, 0)),
            full((8, 128)),
            pl.BlockSpec((ROWS, 1), lambda i: (i, 0)),
        ],
        out_specs=[full((128, 128)), full((G, 128, 128)), full((128, G))],
        out_shape=[
            jax.ShapeDtypeStruct((128, 128), F32),
            jax.ShapeDtypeStruct((G, 128, 128), F32),
            jax.ShapeDtypeStruct((128, G), F32),
        ],
        scratch_shapes=[
            pltpu.VMEM((128, 128), F32),
            pltpu.VMEM((G, 128, 128), F32),
            pltpu.VMEM((128, G), F32),
        ],
    )(a2, x2, cmax, batch_p)


_F3C = 8


def _tc_f3(Ct, Cg, RT, WL3, b_lin):
    def body(ct_ref, cg_ref, rt_ref, wl3, blin, out_ref, acc):
        i = pl.program_id(0)

        @pl.when(i == 0)
        def _():
            acc[...] = jnp.zeros((G, DOUT), F32)

        eye = (lax.broadcasted_iota(jnp.int32, (G, G), 0) ==
               lax.broadcasted_iota(jnp.int32, (G, G), 1)).astype(F32)
        for j in range(_F3C):
            ct_i = ct_ref[j:j + 1, :]
            cg_i = jnp.reshape(cg_ref[:, j, :], (G, 128))
            d_col = lax.dot_general(
                eye, rt_ref[j:j + 1, :], (((1,), (1,)), ((), ())),
                preferred_element_type=F32)
            h_i = (ct_i - cg_i) * d_col
            w_i = jnp.reshape(wl3[j, :, :], (128, DOUT))
            acc[...] = acc[...] + jnp.dot(h_i, w_i,
                                          preferred_element_type=F32)

        @pl.when(i == (128 // _F3C) - 1)
        def _():
            o = acc[...] + blin[...]
            m = jnp.max(o, axis=1, keepdims=True)
            e = jnp.exp(o - m)
            out_ref[...] = e / jnp.sum(e, axis=1, keepdims=True)

    full = lambda s: pl.BlockSpec(s, lambda i: tuple(0 for _ in s))
    return pl.pallas_call(
        body,
        grid=(128 // _F3C,),
        in_specs=[
            pl.BlockSpec((_F3C, 128), lambda i: (i, 0)),
            pl.BlockSpec((G, _F3C, 128), lambda i: (0, i, 0)),
            pl.BlockSpec((_F3C, G), lambda i: (i, 0)),
            pl.BlockSpec((_F3C, 128, DOUT), lambda i: (i, 0, 0)),
            full((1, DOUT)),
        ],
        out_specs=full((G, DOUT)),
        out_shape=jax.ShapeDtypeStruct((G, DOUT), F32),
        scratch_shapes=[pltpu.VMEM((G, DOUT), F32)],
    )(Ct, Cg, RT, WL3, b_lin.reshape(1, DOUT))


def kernel(x, edge_index, batch, num_graphs, W_a1, b_a1, W_a2, b_a2,
           W_x1, b_x1, W_x2, b_x2, W_lin, b_lin):
    del num_graphs
    src = edge_index[0].astype(jnp.int32)
    dst = edge_index[1].astype(jnp.int32)
    pad_e = EPAD - E
    src_p = jnp.concatenate([src, jnp.zeros((pad_e,), jnp.int32)])
    dst_p = jnp.concatenate([dst, jnp.full((pad_e,), N, jnp.int32)])
    x_p = jnp.pad(x, ((0, NPAD - N), (0, 0)))
    batch_p = jnp.pad(batch.astype(jnp.int32), (0, NPAD - N),
                      constant_values=G).reshape(NPAD, 1)
    z128 = jnp.zeros((ROWS, 128), F32)
    ones128 = jnp.ones((K, 128), F32)

    degp = _sc_deg(dst_p, z128, ones128)
    dinv, u = _tc_prep(degp, x_p)
    agg1 = _sc_agg(u, u, src_p, dst_p, z128, split_edges=True)
    ua, ux = _tc_mid(agg1, u, dinv, W_a1, b_a1, W_x1, b_x1, W_a2, W_x2)
    agg2 = _sc_agg(ua, ux, src_p, dst_p, z128, split_edges=False)
    a2, x2, cmax = _tc_f1(agg2, ua, ux, dinv, b_a2, b_x2)
    Ct, Cg, RT = _tc_f2(a2, x2, cmax, batch_p)
    out = _tc_f3(Ct, Cg, RT, W_lin.reshape(128, 128, DOUT), b_lin)
    return out


# speedup vs baseline: 9.3003x; 1.0034x over previous
import functools

import jax
import jax.numpy as jnp
from jax import lax
from jax.experimental import pallas as pl
from jax.experimental.pallas import tpu as pltpu
from jax.experimental.pallas import tpu_sc as plsc

N = 10000
NPAD = 10240
ROWS = NPAD // 16
E = 320000
EPAD = 327680
K = 128
NBS = EPAD // 32 // K
NBF = EPAD // 16 // K
G = 16
D0, D1, D2, DOUT = 128, 256, 128, 10
F32 = jnp.float32

def _mesh():
    return plsc.VectorSubcoreMesh(
        core_axis_name="c", subcore_axis_name="s", num_cores=2,
        num_subcores=16)


def _sc_deg(dst_p, z128, ones128):
    @functools.partial(
        pl.kernel,
        out_type=jax.ShapeDtypeStruct((2 * NPAD, 128), F32),
        mesh=_mesh(),
        scratch_types=[
            pltpu.VMEM((NBS, K), jnp.int32),
            pltpu.VMEM((K, 128), F32),
            pltpu.VMEM_SHARED((NPAD, 128), F32),
        ],
    )
    def k(dst_hbm, z_hbm, ones_hbm, out_hbm, dstbuf, onesbuf, acc):
        cid = lax.axis_index("c")
        sid = lax.axis_index("s")
        row_lo = sid * ROWS
        pltpu.sync_copy(z_hbm, acc.at[pl.ds(row_lo, ROWS)])
        pltpu.sync_copy(ones_hbm, onesbuf)
        bbase = (cid * 16 + sid) * NBS
        pltpu.sync_copy(dst_hbm.at[pl.ds(bbase, NBS)], dstbuf)
        plsc.subcore_barrier()

        def body(j, carry):
            pltpu.sync_copy(onesbuf, acc.at[dstbuf.at[j]], add=True)
            return carry

        lax.fori_loop(0, NBS, body, 0)
        plsc.subcore_barrier()
        pltpu.sync_copy(acc.at[pl.ds(row_lo, ROWS)],
                        out_hbm.at[pl.ds(cid * NPAD + row_lo, ROWS)])

    return k(dst_p, z128, ones128)


def _sc_agg(t0, t1, src_p, dst_p, z128, split_edges):
    nb = NBS if split_edges else NBF
    CH = 16
    nch = nb // CH

    @functools.partial(
        pl.kernel,
        out_type=jax.ShapeDtypeStruct((2 * NPAD, 128), F32),
        mesh=_mesh(),
        scratch_types=[
            pltpu.VMEM((CH, K), jnp.int32),
            pltpu.VMEM((CH, K), jnp.int32),
            pltpu.VMEM((K, 128), F32),
            pltpu.VMEM((K, 128), F32),
            pltpu.VMEM_SHARED((NPAD, 128), F32),
            pltpu.SemaphoreType.DMA,
            pltpu.SemaphoreType.DMA,
        ],
    )
    def k(t0_hbm, t1_hbm, src_hbm, dst_hbm, z_hbm, out_hbm,
          srcbuf, dstbuf, rows0, rows1, acc, sem0, sem1):
        cid = lax.axis_index("c")
        sid = lax.axis_index("s")
        row_lo = sid * ROWS
        pltpu.sync_copy(z_hbm, acc.at[pl.ds(row_lo, ROWS)])
        if split_edges:
            bbase = (cid * 16 + sid) * nb
        else:
            bbase = sid * nb
        plsc.subcore_barrier()

        def gather(j, buf, sem):
            @pl.when(cid == 0)
            def _():
                pltpu.async_copy(t0_hbm.at[srcbuf.at[j]], buf, sem)

            @pl.when(cid == 1)
            def _():
                pltpu.async_copy(t1_hbm.at[srcbuf.at[j]], buf, sem)

        def wait(buf, sem):
            pltpu.make_async_copy(t0_hbm.at[pl.ds(0, K)], buf, sem).wait()

        def chunk(c, carry):
            pltpu.sync_copy(src_hbm.at[pl.ds(bbase + c * CH, CH)], srcbuf)
            pltpu.sync_copy(dst_hbm.at[pl.ds(bbase + c * CH, CH)], dstbuf)
            gather(0, rows0, sem0)

            def body(j2, carry2):
                j = 2 * j2
                gather(j + 1, rows1, sem1)
                wait(rows0, sem0)
                pltpu.sync_copy(rows0, acc.at[dstbuf.at[j]], add=True)

                @pl.when(j2 + 1 < CH // 2)
                def _():
                    gather(j + 2, rows0, sem0)

                wait(rows1, sem1)
                pltpu.sync_copy(rows1, acc.at[dstbuf.at[j + 1]], add=True)
                return carry2

            lax.fori_loop(0, CH // 2, body, 0)
            return carry

        lax.fori_loop(0, nch, chunk, 0)
        plsc.subcore_barrier()
        pltpu.sync_copy(acc.at[pl.ds(row_lo, ROWS)],
                        out_hbm.at[pl.ds(cid * NPAD + row_lo, ROWS)])

    return k(t0, t1, src_p, dst_p, z128)


_NBLK = NPAD // ROWS


def _tc_prep(degp, x_p):
    def body(d0, d1, x_ref, dinv_ref, u_ref):
        deg = d0[:, 0:1] + d1[:, 0:1] + 2.0
        dinv = lax.rsqrt(deg)
        dinv_ref[...] = dinv
        u_ref[...] = dinv * x_ref[...]

    return pl.pallas_call(
        body,
        grid=(_NBLK,),
        in_specs=[
            pl.BlockSpec((ROWS, 128), lambda i: (i, 0)),
            pl.BlockSpec((ROWS, 128), lambda i: (_NBLK + i, 0)),
            pl.BlockSpec((ROWS, 128), lambda i: (i, 0)),
        ],
        out_specs=[
            pl.BlockSpec((ROWS, 1), lambda i: (i, 0)),
            pl.BlockSpec((ROWS, 128), lambda i: (i, 0)),
        ],
        out_shape=[
            jax.ShapeDtypeStruct((NPAD, 1), F32),
            jax.ShapeDtypeStruct((NPAD, 128), F32),
        ],
    )(degp, degp, x_p)


def _tc_mid(agg1, u, dinv, W_a1, b_a1, W_x1, b_x1, W_a2, W_x2):
    def body(p0, p1, u_ref, dinv_ref, wa1, ba1, wx1, bx1, wa2, wx2,
             ua_ref, ux_ref):
        dv = dinv_ref[...]
        px = dv * (p0[...] + p1[...] + 2.0 * u_ref[...])
        a1 = jnp.maximum(
            jnp.dot(px, wa1[...], preferred_element_type=F32) + ba1[...], 0.0)
        x1 = jnp.maximum(
            jnp.dot(px, wx1[...], preferred_element_type=F32) + bx1[...], 0.0)
        ua_ref[...] = dv * jnp.dot(a1, wa2[...], preferred_element_type=F32)
        ux_ref[...] = dv * jnp.dot(x1, wx2[...], preferred_element_type=F32)

    full = lambda s: pl.BlockSpec(s, lambda i: tuple(0 for _ in s))
    return pl.pallas_call(
        body,
        grid=(_NBLK,),
        in_specs=[
            pl.BlockSpec((ROWS, 128), lambda i: (i, 0)),
            pl.BlockSpec((ROWS, 128), lambda i: (_NBLK + i, 0)),
            pl.BlockSpec((ROWS, 128), lambda i: (i, 0)),
            pl.BlockSpec((ROWS, 1), lambda i: (i, 0)),
            full((D0, D1)), full((1, D1)), full((D0, D1)), full((1, D1)),
            full((D1, D2)), full((D1, D2)),
        ],
        out_specs=[
            pl.BlockSpec((ROWS, 128), lambda i: (i, 0)),
            pl.BlockSpec((ROWS, 128), lambda i: (i, 0)),
        ],
        out_shape=[
            jax.ShapeDtypeStruct((NPAD, 128), F32),
            jax.ShapeDtypeStruct((NPAD, 128), F32),
        ],
    )(agg1, agg1, u, dinv, W_a1, b_a1.reshape(1, D1), W_x1,
      b_x1.reshape(1, D1), W_a2, W_x2)


def _tc_f1(agg2, ua, ux, dinv, b_a2, b_x2):
    def body(q0, q1, ua_ref, ux_ref, dinv_ref, ba2, bx2,
             a2_ref, x2_ref, cmax_ref):
        i = pl.program_id(0)
        dv = dinv_ref[...]
        a2 = dv * (q0[...] + 2.0 * ua_ref[...]) + ba2[...]
        x2 = jnp.maximum(dv * (q1[...] + 2.0 * ux_ref[...]) + bx2[...], 0.0)
        rid = i * ROWS + lax.broadcasted_iota(jnp.int32, (ROWS, 1), 0)
        a2 = jnp.where(rid < N, a2, -1e30)
        a2_ref[...] = a2
        x2_ref[...] = x2

        @pl.when(i == 0)
        def _():
            cmax_ref[...] = jnp.full((8, 128), -1e30, F32)

        bm = jnp.max(a2, axis=0, keepdims=True)
        cmax_ref[0:1, :] = jnp.maximum(cmax_ref[0:1, :], bm)

    full = lambda s: pl.BlockSpec(s, lambda i: tuple(0 for _ in s))
    return pl.pallas_call(
        body,
        grid=(_NBLK,),
        in_specs=[
            pl.BlockSpec((ROWS, 128), lambda i: (i, 0)),
            pl.BlockSpec((ROWS, 128), lambda i: (_NBLK + i, 0)),
            pl.BlockSpec((ROWS, 128), lambda i: (i, 0)),
            pl.BlockSpec((ROWS, 128), lambda i: (i, 0)),
            pl.BlockSpec((ROWS, 1), lambda i: (i, 0)),
            full((1, D2)), full((1, D2)),
        ],
        out_specs=[
            pl.BlockSpec((ROWS, 128), lambda i: (i, 0)),
            pl.BlockSpec((ROWS, 128), lambda i: (i, 0)),
            full((8, 128)),
        ],
        out_shape=[
            jax.ShapeDtypeStruct((NPAD, 128), F32),
            jax.ShapeDtypeStruct((NPAD, 128), F32),
            jax.ShapeDtypeStruct((8, 128), F32),
        ],
    )(agg2, agg2, ua, ux, dinv, b_a2.reshape(1, D2), b_x2.reshape(1, D2))


def _tc_f2(a2, x2, cmax, batch_p):
    def body(a2_ref, x2_ref, cmax_ref, bat_ref, ct_out, cg_out, rt_out,
             Ct, Cg, SgT):
        i = pl.program_id(0)

        @pl.when(i == 0)
        def _():
            Ct[...] = jnp.zeros((128, 128), F32)
            Cg[...] = jnp.zeros((G, 128, 128), F32)
            SgT[...] = jnp.zeros((128, G), F32)

        U = jnp.exp(a2_ref[...] - cmax_ref[0:1, :])
        x2b = x2_ref[...]
        Ct[...] = Ct[...] + lax.dot_general(
            U, x2b, (((0,), (0,)), ((), ())), preferred_element_type=F32)
        oh = (bat_ref[...] ==
              lax.broadcasted_iota(jnp.int32, (1, G), 1)).astype(F32)
        SgT[...] = SgT[...] + lax.dot_general(
            U, oh, (((0,), (0,)), ((), ())), preferred_element_type=F32)
        for g in range(G):
            Ug = U * oh[:, g:g + 1]
            Cg[g] = Cg[g] + lax.dot_general(
                Ug, x2b, (((0,), (0,)), ((), ())), preferred_element_type=F32)

        @pl.when(i == _NBLK - 1)
        def _():
            st = jnp.sum(SgT[...], axis=1, keepdims=True)
            rt_out[...] = 1.0 / (st - SgT[...])
            ct_out[...] = Ct[...]
            cg_out[...] = Cg[...]

    full = lambda s: pl.BlockSpec(s, lambda i: tuple(0 for _ in s))
    return pl.pallas_call(
        body,
        grid=(_NBLK,),
        in_specs=[
            pl.BlockSpec((ROWS, 128), lambda i: (i, 0)),
            pl.BlockSpec((ROWS, 128), lambda i: (i, 0)),
            full((8, 128)),
            pl.BlockSpec((ROWS, 1), lambda i: (i, 0)),
        ],
        out_specs=[full((128, 128)), full((G, 128, 128)), full((128, G))],
        out_shape=[
            jax.ShapeDtypeStruct((128, 128), F32),
            jax.ShapeDtypeStruct((G, 128, 128), F32),
            jax.ShapeDtypeStruct((128, G), F32),
        ],
        scratch_shapes=[
            pltpu.VMEM((128, 128), F32),
            pltpu.VMEM((G, 128, 128), F32),
            pltpu.VMEM((128, G), F32),
        ],
    )(a2, x2, cmax, batch_p)


_F3C = 8


def _tc_f3(Ct, Cg, RT, WL3, b_lin):
    def body(ct_ref, cg_ref, rt_ref, wl3, blin, out_ref, acc):
        i = pl.program_id(0)

        @pl.when(i == 0)
        def _():
            acc[...] = jnp.zeros((G, DOUT), F32)

        eye = (lax.broadcasted_iota(jnp.int32, (G, G), 0) ==
               lax.broadcasted_iota(jnp.int32, (G, G), 1)).astype(F32)
        for j in range(_F3C):
            ct_i = ct_ref[j:j + 1, :]
            cg_i = jnp.reshape(cg_ref[:, j, :], (G, 128))
            d_col = lax.dot_general(
                eye, rt_ref[j:j + 1, :], (((1,), (1,)), ((), ())),
                preferred_element_type=F32)
            h_i = (ct_i - cg_i) * d_col
            w_i = jnp.reshape(wl3[j, :, :], (128, DOUT))
            acc[...] = acc[...] + jnp.dot(h_i, w_i,
                                          preferred_element_type=F32)

        @pl.when(i == (128 // _F3C) - 1)
        def _():
            o = acc[...] + blin[...]
            m = jnp.max(o, axis=1, keepdims=True)
            e = jnp.exp(o - m)
            out_ref[...] = e / jnp.sum(e, axis=1, keepdims=True)

    full = lambda s: pl.BlockSpec(s, lambda i: tuple(0 for _ in s))
    return pl.pallas_call(
        body,
        grid=(128 // _F3C,),
        in_specs=[
            pl.BlockSpec((_F3C, 128), lambda i: (i, 0)),
            pl.BlockSpec((G, _F3C, 128), lambda i: (0, i, 0)),
            pl.BlockSpec((_F3C, G), lambda i: (i, 0)),
            pl.BlockSpec((_F3C, 128, DOUT), lambda i: (i, 0, 0)),
            full((1, DOUT)),
        ],
        out_specs=full((G, DOUT)),
        out_shape=jax.ShapeDtypeStruct((G, DOUT), F32),
        scratch_shapes=[pltpu.VMEM((G, DOUT), F32)],
    )(Ct, Cg, RT, WL3, b_lin.reshape(1, DOUT))


def kernel(x, edge_index, batch, num_graphs, W_a1, b_a1, W_a2, b_a2,
           W_x1, b_x1, W_x2, b_x2, W_lin, b_lin):
    del num_graphs
    src = edge_index[0].astype(jnp.int32)
    dst = edge_index[1].astype(jnp.int32)
    pad_e = EPAD - E
    src_p = jnp.concatenate(
        [src, jnp.zeros((pad_e,), jnp.int32)]).reshape(EPAD // K, K)
    dst_p = jnp.concatenate(
        [dst, jnp.full((pad_e,), N, jnp.int32)]).reshape(EPAD // K, K)
    x_p = jnp.pad(x, ((0, NPAD - N), (0, 0)))
    batch_p = jnp.pad(batch.astype(jnp.int32), (0, NPAD - N),
                      constant_values=G).reshape(NPAD, 1)
    z128 = jnp.zeros((ROWS, 128), F32)
    ones128 = jnp.ones((K, 128), F32)

    degp = _sc_deg(dst_p, z128, ones128)
    dinv, u = _tc_prep(degp, x_p)
    agg1 = _sc_agg(u, u, src_p, dst_p, z128, split_edges=True)
    ua, ux = _tc_mid(agg1, u, dinv, W_a1, b_a1, W_x1, b_x1, W_a2, W_x2)
    agg2 = _sc_agg(ua, ux, src_p, dst_p, z128, split_edges=False)
    a2, x2, cmax = _tc_f1(agg2, ua, ux, dinv, b_a2, b_x2)
    Ct, Cg, RT = _tc_f2(a2, x2, cmax, batch_p)
    out = _tc_f3(Ct, Cg, RT, W_lin.reshape(128, 128, DOUT), b_lin)
    return out
